# baseline (speedup 1.0000x reference)
import jax
import jax.numpy as jnp
from jax import lax
from jax.experimental import pallas as pl
from jax.experimental.pallas import tpu as pltpu
from jax.experimental.pallas import tpu_sc as plsc

N = 10000
E = 320000
D = 128
NS = 16
EPT = E // NS
B = 128
NBT = 160
EPTP = NBT * B
IC = 16
NCH = NBT // IC
NPAD = 10112
RPT = NPAD // NS
CW = 16
PAD_ROW = 10100


def _ln_relu_body(xs_ref, xd_ref, gs_ref, bs_ref, gd_ref, bd_ref,
                  hs_ref, hd_ref):
    for x_ref, g_ref, b_ref, h_ref in (
        (xs_ref, gs_ref, bs_ref, hs_ref),
        (xd_ref, gd_ref, bd_ref, hd_ref),
    ):
        x = x_ref[...]
        m = jnp.mean(x, axis=-1, keepdims=True)
        v = jnp.mean(jnp.square(x - m), axis=-1, keepdims=True)
        h = (x - m) * jax.lax.rsqrt(v + 1e-5) * g_ref[...] + b_ref[...]
        h_ref[...] = jnp.maximum(h, 0.0)


def _sc_agg_body(hs_ref, hd_ref, ei_ref, z_rows_ref, z_cnt_ref, one_cnt_ref,
                 msg_d_ref, cnt_d_ref, msg_s_ref, cnt_s_ref,
                 acc_msg, acc_cnt, gidx, sidx, rows, ones,
                 sem_g0, sem_g1, sem_s0, sem_s1):
    s = lax.axis_index("s")
    sem_g = (sem_g0, sem_g1)
    sem_s = (sem_s0, sem_s1)

    def run(h_ref, gsel, ssel, msg_out, cnt_out):
        pltpu.sync_copy(one_cnt_ref, ones)
        pltpu.sync_copy(z_rows_ref, acc_msg.at[pl.ds(s * RPT, RPT)])
        pltpu.sync_copy(z_cnt_ref, acc_cnt.at[pl.ds(s * RPT, RPT)])
        plsc.subcore_barrier()

        def gather(k, b):
            return pltpu.async_copy(
                h_ref.at[gidx.at[k]], rows.at[b], sem_g[b])

        pltpu.sync_copy(ei_ref.at[gsel, s, 0], gidx)
        pltpu.sync_copy(ei_ref.at[ssel, s, 0], sidx)
        gather(0, 0)

        def chunk(ci, carry):
            cnt_dmas = []
            for k in range(IC):
                b = k % 2
                nb = 1 - b
                pltpu.make_async_copy(
                    h_ref.at[gidx.at[k]], rows.at[b], sem_g[b]).wait()
                if k < IC - 1:
                    gather(k + 1, nb)
                pltpu.sync_copy(rows.at[b], acc_msg.at[sidx.at[k]], add=True)
                cnt_dmas.append(pltpu.async_copy(
                    ones, acc_cnt.at[sidx.at[k]], sem_s[0], add=True))
            for d in cnt_dmas:
                d.wait()

            @pl.when(ci < NCH - 1)
            def _():
                pltpu.sync_copy(ei_ref.at[gsel, s, ci + 1], gidx)
                pltpu.sync_copy(ei_ref.at[ssel, s, ci + 1], sidx)
                gather(0, 0)

            return carry

        lax.fori_loop(0, NCH, chunk, 0)
        plsc.subcore_barrier()
        sl = pl.ds(s * RPT, RPT)
        pltpu.sync_copy(acc_msg.at[sl], msg_out.at[sl])
        pltpu.sync_copy(acc_cnt.at[sl], cnt_out.at[sl])

    c = lax.axis_index("c")

    @pl.when(c == 0)
    def _():
        run(hs_ref, 0, 1, msg_d_ref, cnt_d_ref)

    @pl.when(c == 1)
    def _():
        run(hd_ref, 2, 3, msg_s_ref, cnt_s_ref)


def _combine_body(xs_ref, xd_ref, hs_ref, hd_ref,
                  msg_d_ref, cnt_d_ref, msg_s_ref, cnt_s_ref,
                  wss_ref, wns_ref, wsd_ref, wnd_ref,
                  ys_ref, yd_ref):
    inv_d = 1.0 / jnp.maximum(cnt_d_ref[:, :1], 1.0)
    inv_s = 1.0 / jnp.maximum(cnt_s_ref[:, :1], 1.0)
    f32 = jnp.float32
    yd = jnp.dot(hd_ref[...], wsd_ref[...], preferred_element_type=f32)
    yd += jnp.dot(msg_d_ref[...], wnd_ref[...], preferred_element_type=f32) * inv_d
    yd_ref[...] = xd_ref[...] + yd
    ys = jnp.dot(hs_ref[...], wss_ref[...], preferred_element_type=f32)
    ys += jnp.dot(msg_s_ref[...], wns_ref[...], preferred_element_type=f32) * inv_s
    ys_ref[...] = xs_ref[...] + ys


def _make_sc_agg():
    mesh = plsc.VectorSubcoreMesh(core_axis_name="c", subcore_axis_name="s")
    return pl.kernel(
        _sc_agg_body,
        out_type=(
            jax.ShapeDtypeStruct((NPAD, D), jnp.float32),
            jax.ShapeDtypeStruct((NPAD, CW), jnp.float32),
            jax.ShapeDtypeStruct((NPAD, D), jnp.float32),
            jax.ShapeDtypeStruct((NPAD, CW), jnp.float32),
        ),
        mesh=mesh,
        compiler_params=pltpu.CompilerParams(use_tc_tiling_on_sc=False),
        scratch_types=[
            pltpu.VMEM_SHARED((NPAD, D), jnp.float32),
            pltpu.VMEM_SHARED((NPAD, CW), jnp.float32),
            pltpu.VMEM((IC, B), jnp.int32),
            pltpu.VMEM((IC, B), jnp.int32),
            pltpu.VMEM((2, B, D), jnp.float32),
            pltpu.VMEM((B, CW), jnp.float32),
            pltpu.SemaphoreType.DMA,
            pltpu.SemaphoreType.DMA,
            pltpu.SemaphoreType.DMA,
            pltpu.SemaphoreType.DMA,
        ],
    )


def kernel(x_src, x_dst, ln_g_src, ln_b_src, ln_g_dst, ln_b_dst,
           W_self_src, W_nbr_src, W_self_dst, W_nbr_dst, edge_index):
    f32 = jnp.float32
    i32 = jnp.int32
    RB = 2000
    G = N // RB

    gs = ln_g_src.reshape(1, D)
    bs = ln_b_src.reshape(1, D)
    gd = ln_g_dst.reshape(1, D)
    bd = ln_b_dst.reshape(1, D)

    row_spec = pl.BlockSpec((RB, D), lambda i: (i, 0))
    vec_spec = pl.BlockSpec((1, D), lambda i: (0, 0))
    h_src, h_dst = pl.pallas_call(
        _ln_relu_body,
        grid=(G,),
        in_specs=[row_spec, row_spec, vec_spec, vec_spec, vec_spec, vec_spec],
        out_specs=[row_spec, row_spec],
        out_shape=[jax.ShapeDtypeStruct((N, D), f32)] * 2,
    )(x_src, x_dst, gs, bs, gd, bd)

    ei = edge_index.reshape(2, NS, EPT)
    pad_g = jnp.zeros((NS, EPTP - EPT), i32)
    pad_s = jnp.full((NS, EPTP - EPT), PAD_ROW, i32)
    eis = jnp.stack([
        jnp.concatenate([ei[0], pad_g], axis=-1),
        jnp.concatenate([ei[1], pad_s], axis=-1),
        jnp.concatenate([ei[1], pad_g], axis=-1),
        jnp.concatenate([ei[0], pad_s], axis=-1),
    ]).reshape(4, NS, NCH, IC, B)

    z_rows = jnp.zeros((RPT, D), f32)
    z_cnt = jnp.zeros((RPT, CW), f32)
    one_cnt = jnp.ones((B, CW), f32)
    msg_d, cnt_d, msg_s, cnt_s = _make_sc_agg()(
        h_src, h_dst, eis, z_rows, z_cnt, one_cnt)

    cnt_spec = pl.BlockSpec((RB, CW), lambda i: (i, 0))
    w_spec = pl.BlockSpec((D, D), lambda i: (0, 0))
    y_src, y_dst = pl.pallas_call(
        _combine_body,
        grid=(G,),
        in_specs=[row_spec, row_spec, row_spec, row_spec,
                  row_spec, cnt_spec, row_spec, cnt_spec,
                  w_spec, w_spec, w_spec, w_spec],
        out_specs=[row_spec, row_spec],
        out_shape=[jax.ShapeDtypeStruct((N, D), f32)] * 2,
    )(x_src, x_dst, h_src, h_dst,
      msg_d, cnt_d, msg_s, cnt_s,
      W_self_src, W_nbr_src, W_self_dst, W_nbr_dst)

    return (y_src, y_dst)


# speedup vs baseline: 1.0643x; 1.0643x over previous
import jax
import jax.numpy as jnp
from jax import lax
from jax.experimental import pallas as pl
from jax.experimental.pallas import tpu as pltpu
from jax.experimental.pallas import tpu_sc as plsc

N = 10000
E = 320000
D = 128
NS = 16
EPT = E // NS
B = 128
NBT = 160
EPTP = NBT * B
IC = 16
NCH = NBT // IC
NPAD = 10112
RPT = NPAD // NS
CW = 16
PAD_ROW = 10100


def _ln_relu_body(xs_ref, xd_ref, gs_ref, bs_ref, gd_ref, bd_ref,
                  hs_ref, hd_ref):
    for x_ref, g_ref, b_ref, h_ref in (
        (xs_ref, gs_ref, bs_ref, hs_ref),
        (xd_ref, gd_ref, bd_ref, hd_ref),
    ):
        x = x_ref[...]
        m = jnp.mean(x, axis=-1, keepdims=True)
        v = jnp.mean(jnp.square(x - m), axis=-1, keepdims=True)
        h = (x - m) * jax.lax.rsqrt(v + 1e-5) * g_ref[...] + b_ref[...]
        h_ref[...] = jnp.maximum(h, 0.0)


def _sc_agg_body(hs_ref, hd_ref, ei_ref, z_rows_ref, z_cnt_ref, one_cnt_ref,
                 msg_d_ref, cnt_d_ref, msg_s_ref, cnt_s_ref,
                 acc_msg, acc_cnt, gidx, sidx, rows, ones,
                 sem_g0, sem_g1, sem_s0, sem_s1):
    s = lax.axis_index("s")
    sem_g = (sem_g0, sem_g1)
    sem_s = (sem_s0, sem_s1)

    def run(h_ref, gsel, ssel, msg_out, cnt_out):
        pltpu.sync_copy(one_cnt_ref, ones)
        pltpu.sync_copy(z_rows_ref, acc_msg.at[pl.ds(s * RPT, RPT)])
        pltpu.sync_copy(z_cnt_ref, acc_cnt.at[pl.ds(s * RPT, RPT)])
        plsc.subcore_barrier()

        def gather(k, b):
            return pltpu.async_copy(
                h_ref.at[gidx.at[k]], rows.at[b], sem_g[b])

        def chunk(ci, carry):
            pltpu.sync_copy(ei_ref.at[gsel, s, ci], gidx)
            pltpu.sync_copy(ei_ref.at[ssel, s, ci], sidx)
            g = {0: gather(0, 0), 1: gather(1, 1)}
            cnt_dmas = []
            for k in range(IC):
                b = k % 2
                g.pop(k).wait()
                pltpu.sync_copy(rows.at[b], acc_msg.at[sidx.at[k]], add=True)
                cnt_dmas.append(pltpu.async_copy(
                    ones, acc_cnt.at[sidx.at[k]], sem_s[0], add=True))
                if k + 2 < IC:
                    g[k + 2] = gather(k + 2, b)
            for d in cnt_dmas:
                d.wait()
            return carry

        lax.fori_loop(0, NCH, chunk, 0)
        plsc.subcore_barrier()
        sl = pl.ds(s * RPT, RPT)
        pltpu.sync_copy(acc_msg.at[sl], msg_out.at[sl])
        pltpu.sync_copy(acc_cnt.at[sl], cnt_out.at[sl])

    c = lax.axis_index("c")

    @pl.when(c == 0)
    def _():
        run(hs_ref, 0, 1, msg_d_ref, cnt_d_ref)

    @pl.when(c == 1)
    def _():
        run(hd_ref, 2, 3, msg_s_ref, cnt_s_ref)


def _combine_body(xs_ref, xd_ref, hs_ref, hd_ref,
                  msg_d_ref, cnt_d_ref, msg_s_ref, cnt_s_ref,
                  wss_ref, wns_ref, wsd_ref, wnd_ref,
                  ys_ref, yd_ref):
    inv_d = 1.0 / jnp.maximum(cnt_d_ref[:, :1], 1.0)
    inv_s = 1.0 / jnp.maximum(cnt_s_ref[:, :1], 1.0)
    f32 = jnp.float32
    yd = jnp.dot(hd_ref[...], wsd_ref[...], preferred_element_type=f32)
    yd += jnp.dot(msg_d_ref[...], wnd_ref[...], preferred_element_type=f32) * inv_d
    yd_ref[...] = xd_ref[...] + yd
    ys = jnp.dot(hs_ref[...], wss_ref[...], preferred_element_type=f32)
    ys += jnp.dot(msg_s_ref[...], wns_ref[...], preferred_element_type=f32) * inv_s
    ys_ref[...] = xs_ref[...] + ys


def _make_sc_agg():
    mesh = plsc.VectorSubcoreMesh(core_axis_name="c", subcore_axis_name="s")
    return pl.kernel(
        _sc_agg_body,
        out_type=(
            jax.ShapeDtypeStruct((NPAD, D), jnp.float32),
            jax.ShapeDtypeStruct((NPAD, CW), jnp.float32),
            jax.ShapeDtypeStruct((NPAD, D), jnp.float32),
            jax.ShapeDtypeStruct((NPAD, CW), jnp.float32),
        ),
        mesh=mesh,
        compiler_params=pltpu.CompilerParams(use_tc_tiling_on_sc=False),
        scratch_types=[
            pltpu.VMEM_SHARED((NPAD, D), jnp.float32),
            pltpu.VMEM_SHARED((NPAD, CW), jnp.float32),
            pltpu.VMEM((IC, B), jnp.int32),
            pltpu.VMEM((IC, B), jnp.int32),
            pltpu.VMEM((2, B, D), jnp.float32),
            pltpu.VMEM((B, CW), jnp.float32),
            pltpu.SemaphoreType.DMA,
            pltpu.SemaphoreType.DMA,
            pltpu.SemaphoreType.DMA,
            pltpu.SemaphoreType.DMA,
        ],
    )


def kernel(x_src, x_dst, ln_g_src, ln_b_src, ln_g_dst, ln_b_dst,
           W_self_src, W_nbr_src, W_self_dst, W_nbr_dst, edge_index):
    f32 = jnp.float32
    i32 = jnp.int32
    RB = 2000
    G = N // RB

    gs = ln_g_src.reshape(1, D)
    bs = ln_b_src.reshape(1, D)
    gd = ln_g_dst.reshape(1, D)
    bd = ln_b_dst.reshape(1, D)

    row_spec = pl.BlockSpec((RB, D), lambda i: (i, 0))
    vec_spec = pl.BlockSpec((1, D), lambda i: (0, 0))
    h_src, h_dst = pl.pallas_call(
        _ln_relu_body,
        grid=(G,),
        in_specs=[row_spec, row_spec, vec_spec, vec_spec, vec_spec, vec_spec],
        out_specs=[row_spec, row_spec],
        out_shape=[jax.ShapeDtypeStruct((N, D), f32)] * 2,
    )(x_src, x_dst, gs, bs, gd, bd)

    ei = edge_index.reshape(2, NS, EPT)
    pad_g = jnp.zeros((NS, EPTP - EPT), i32)
    pad_s = jnp.full((NS, EPTP - EPT), PAD_ROW, i32)
    eis = jnp.stack([
        jnp.concatenate([ei[0], pad_g], axis=-1),
        jnp.concatenate([ei[1], pad_s], axis=-1),
        jnp.concatenate([ei[1], pad_g], axis=-1),
        jnp.concatenate([ei[0], pad_s], axis=-1),
    ]).reshape(4, NS, NCH, IC, B)

    z_rows = jnp.zeros((RPT, D), f32)
    z_cnt = jnp.zeros((RPT, CW), f32)
    one_cnt = jnp.ones((B, CW), f32)
    msg_d, cnt_d, msg_s, cnt_s = _make_sc_agg()(
        h_src, h_dst, eis, z_rows, z_cnt, one_cnt)

    cnt_spec = pl.BlockSpec((RB, CW), lambda i: (i, 0))
    w_spec = pl.BlockSpec((D, D), lambda i: (0, 0))
    y_src, y_dst = pl.pallas_call(
        _combine_body,
        grid=(G,),
        in_specs=[row_spec, row_spec, row_spec, row_spec,
                  row_spec, cnt_spec, row_spec, cnt_spec,
                  w_spec, w_spec, w_spec, w_spec],
        out_specs=[row_spec, row_spec],
        out_shape=[jax.ShapeDtypeStruct((N, D), f32)] * 2,
    )(x_src, x_dst, h_src, h_dst,
      msg_d, cnt_d, msg_s, cnt_s,
      W_self_src, W_nbr_src, W_self_dst, W_nbr_dst)

    return (y_src, y_dst)


# speedup vs baseline: 1.8739x; 1.7607x over previous
import jax
import jax.numpy as jnp
from jax import lax
from jax.experimental import pallas as pl
from jax.experimental.pallas import tpu as pltpu
from jax.experimental.pallas import tpu_sc as plsc

N = 10000
E = 320000
D = 128
NS = 16
EPT = E // NS
B = 80
NBT = 250
EPTP = NBT * B
IC = 10
NCH = NBT // IC
NPAD = 10240
RPT = NPAD // NS
CW = 16
PAD_ROW = 10100


def _ln_relu_body(xs_ref, xd_ref, gs_ref, bs_ref, gd_ref, bd_ref,
                  hs_ref, hd_ref):
    for x_ref, g_ref, b_ref, h_ref in (
        (xs_ref, gs_ref, bs_ref, hs_ref),
        (xd_ref, gd_ref, bd_ref, hd_ref),
    ):
        x = x_ref[...]
        m = jnp.mean(x, axis=-1, keepdims=True)
        v = jnp.mean(jnp.square(x - m), axis=-1, keepdims=True)
        h = (x - m) * jax.lax.rsqrt(v + 1e-5) * g_ref[...] + b_ref[...]
        h_ref[...] = jnp.maximum(h, 0.0)


def _sc_agg_body(hs_ref, hd_ref, ei_ref, z_rows_ref, z_cnt_ref, one_cnt_ref,
                 msg_d_ref, cnt_d_ref, msg_s_ref, cnt_s_ref,
                 acc_msg, acc_cnt, gidx, sidx, rows, ones,
                 sem_g0, sem_g1, sem_s0, sem_s1):
    s = lax.axis_index("s")
    sem_g = (sem_g0, sem_g1)
    sem_s = (sem_s0, sem_s1)

    def run(h_ref, gsel, ssel, msg_out, cnt_out):
        pltpu.sync_copy(one_cnt_ref, ones)
        pltpu.sync_copy(z_rows_ref, acc_msg.at[pl.ds(s * RPT, RPT)])
        pltpu.sync_copy(z_cnt_ref, acc_cnt.at[pl.ds(s * RPT, RPT)])
        plsc.subcore_barrier()

        def gather(k, b):
            return pltpu.async_copy(
                h_ref.at[gidx.at[k]], rows.at[b], sem_g[b])

        def chunk(ci, carry):
            pltpu.sync_copy(ei_ref.at[gsel, s, ci], gidx)
            pltpu.sync_copy(ei_ref.at[ssel, s, ci], sidx)
            g = {0: gather(0, 0), 1: gather(1, 1)}
            cnt_dmas = []
            for k in range(IC):
                b = k % 2
                g.pop(k).wait()
                pltpu.sync_copy(rows.at[b], acc_msg.at[sidx.at[k]], add=True)
                cnt_dmas.append(pltpu.async_copy(
                    ones, acc_cnt.at[sidx.at[k]], sem_s[0], add=True))
                if k + 2 < IC:
                    g[k + 2] = gather(k + 2, b)
            for d in cnt_dmas:
                d.wait()
            return carry

        lax.fori_loop(0, NCH, chunk, 0)
        plsc.subcore_barrier()
        sl = pl.ds(s * RPT, RPT)
        pltpu.sync_copy(acc_msg.at[sl], msg_out.at[sl])
        pltpu.sync_copy(acc_cnt.at[sl], cnt_out.at[sl])

    c = lax.axis_index("c")

    @pl.when(c == 0)
    def _():
        run(hs_ref, 0, 1, msg_d_ref, cnt_d_ref)

    @pl.when(c == 1)
    def _():
        run(hd_ref, 1, 0, msg_s_ref, cnt_s_ref)


def _combine_body(xs_ref, xd_ref, hs_ref, hd_ref,
                  msg_d_ref, cnt_d_ref, msg_s_ref, cnt_s_ref,
                  wss_ref, wns_ref, wsd_ref, wnd_ref,
                  ys_ref, yd_ref):
    inv_d = 1.0 / jnp.maximum(cnt_d_ref[:, :1], 1.0)
    inv_s = 1.0 / jnp.maximum(cnt_s_ref[:, :1], 1.0)
    f32 = jnp.float32
    yd = jnp.dot(hd_ref[...], wsd_ref[...], preferred_element_type=f32)
    yd += jnp.dot(msg_d_ref[...], wnd_ref[...], preferred_element_type=f32) * inv_d
    yd_ref[...] = xd_ref[...] + yd
    ys = jnp.dot(hs_ref[...], wss_ref[...], preferred_element_type=f32)
    ys += jnp.dot(msg_s_ref[...], wns_ref[...], preferred_element_type=f32) * inv_s
    ys_ref[...] = xs_ref[...] + ys


def _make_sc_agg():
    mesh = plsc.VectorSubcoreMesh(core_axis_name="c", subcore_axis_name="s")
    return pl.kernel(
        _sc_agg_body,
        out_type=(
            jax.ShapeDtypeStruct((NPAD, D), jnp.float32),
            jax.ShapeDtypeStruct((NPAD, CW), jnp.float32),
            jax.ShapeDtypeStruct((NPAD, D), jnp.float32),
            jax.ShapeDtypeStruct((NPAD, CW), jnp.float32),
        ),
        mesh=mesh,
        compiler_params=pltpu.CompilerParams(use_tc_tiling_on_sc=False),
        scratch_types=[
            pltpu.VMEM_SHARED((NPAD, D), jnp.float32),
            pltpu.VMEM_SHARED((NPAD, CW), jnp.float32),
            pltpu.VMEM((IC, B), jnp.int32),
            pltpu.VMEM((IC, B), jnp.int32),
            pltpu.VMEM((2, B, D), jnp.float32),
            pltpu.VMEM((B, CW), jnp.float32),
            pltpu.SemaphoreType.DMA,
            pltpu.SemaphoreType.DMA,
            pltpu.SemaphoreType.DMA,
            pltpu.SemaphoreType.DMA,
        ],
    )


def kernel(x_src, x_dst, ln_g_src, ln_b_src, ln_g_dst, ln_b_dst,
           W_self_src, W_nbr_src, W_self_dst, W_nbr_dst, edge_index):
    f32 = jnp.float32
    i32 = jnp.int32
    RB = 2000
    G = N // RB

    gs = ln_g_src.reshape(1, D)
    bs = ln_b_src.reshape(1, D)
    gd = ln_g_dst.reshape(1, D)
    bd = ln_b_dst.reshape(1, D)

    row_spec = pl.BlockSpec((RB, D), lambda i: (i, 0))
    vec_spec = pl.BlockSpec((1, D), lambda i: (0, 0))
    h_src, h_dst = pl.pallas_call(
        _ln_relu_body,
        grid=(G,),
        in_specs=[row_spec, row_spec, vec_spec, vec_spec, vec_spec, vec_spec],
        out_specs=[row_spec, row_spec],
        out_shape=[jax.ShapeDtypeStruct((N, D), f32)] * 2,
    )(x_src, x_dst, gs, bs, gd, bd)

    eis = edge_index.reshape(2, NS, NCH, IC, B)

    z_rows = jnp.zeros((RPT, D), f32)
    z_cnt = jnp.zeros((RPT, CW), f32)
    one_cnt = jnp.ones((B, CW), f32)
    msg_d, cnt_d, msg_s, cnt_s = _make_sc_agg()(
        h_src, h_dst, eis, z_rows, z_cnt, one_cnt)

    cnt_spec = pl.BlockSpec((RB, CW), lambda i: (i, 0))
    w_spec = pl.BlockSpec((D, D), lambda i: (0, 0))
    y_src, y_dst = pl.pallas_call(
        _combine_body,
        grid=(G,),
        in_specs=[row_spec, row_spec, row_spec, row_spec,
                  row_spec, cnt_spec, row_spec, cnt_spec,
                  w_spec, w_spec, w_spec, w_spec],
        out_specs=[row_spec, row_spec],
        out_shape=[jax.ShapeDtypeStruct((N, D), f32)] * 2,
    )(x_src, x_dst, h_src, h_dst,
      msg_d, cnt_d, msg_s, cnt_s,
      W_self_src, W_nbr_src, W_self_dst, W_nbr_dst)

    return (y_src, y_dst)


# speedup vs baseline: 2.0213x; 1.0787x over previous
import jax
import jax.numpy as jnp
from jax import lax
from jax.experimental import pallas as pl
from jax.experimental.pallas import tpu as pltpu
from jax.experimental.pallas import tpu_sc as plsc

N = 10000
E = 320000
D = 128
NS = 16
EPT = E // NS
B = 80
NBT = 250
EPTP = NBT * B
IC = 10
NCH = NBT // IC
NPAD = 10240
RPT = NPAD // NS
CW = 16
PAD_ROW = 10100


def _ln_relu_body(xs_ref, xd_ref, gs_ref, bs_ref, gd_ref, bd_ref,
                  hs_ref, hd_ref):
    for x_ref, g_ref, b_ref, h_ref in (
        (xs_ref, gs_ref, bs_ref, hs_ref),
        (xd_ref, gd_ref, bd_ref, hd_ref),
    ):
        x = x_ref[...]
        m = jnp.mean(x, axis=-1, keepdims=True)
        v = jnp.mean(jnp.square(x - m), axis=-1, keepdims=True)
        h = (x - m) * jax.lax.rsqrt(v + 1e-5) * g_ref[...] + b_ref[...]
        h_ref[...] = jnp.maximum(h, 0.0)


def _sc_agg_body(hs_ref, hd_ref, ei_ref, z_rows_ref, z_cnt_ref, one_cnt_ref,
                 msg_d_ref, cnt_d_ref, msg_s_ref, cnt_s_ref,
                 acc_msg, acc_cnt, gidx, sidx, rows, ones,
                 sem_g0, sem_g1, sem_s0, sem_s1):
    s = lax.axis_index("s")
    sem_g = (sem_g0, sem_g1)
    sem_s = (sem_s0, sem_s1)

    def run(h_ref, gsel, ssel, msg_out, cnt_out):
        pltpu.sync_copy(one_cnt_ref, ones)
        pltpu.sync_copy(z_rows_ref, acc_msg.at[pl.ds(s * RPT, RPT)])
        pltpu.sync_copy(z_cnt_ref, acc_cnt.at[pl.ds(s * RPT, RPT)])
        plsc.subcore_barrier()

        def gather(k, b):
            return pltpu.async_copy(
                h_ref.at[gidx.at[k]], rows.at[b], sem_g[b % 2])

        def chunk(ci, carry):
            pltpu.sync_copy(ei_ref.at[gsel, s, ci], gidx)
            pltpu.sync_copy(ei_ref.at[ssel, s, ci], sidx)
            g = {0: gather(0, 0), 1: gather(1, 1), 2: gather(2, 2)}
            sc = {}
            cnt_dmas = []
            for k in range(IC):
                b = k % 3
                g.pop(k).wait()
                sc[k] = pltpu.async_copy(
                    rows.at[b], acc_msg.at[sidx.at[k]], sem_s[0], add=True)
                cnt_dmas.append(pltpu.async_copy(
                    ones, acc_cnt.at[sidx.at[k]], sem_s[1], add=True))
                if k >= 1 and k + 2 < IC:
                    sc.pop(k - 1).wait()
                    g[k + 2] = gather(k + 2, (k + 2) % 3)
            for kk in sorted(sc):
                sc[kk].wait()
            for d in cnt_dmas:
                d.wait()
            return carry

        lax.fori_loop(0, NCH, chunk, 0)
        plsc.subcore_barrier()
        sl = pl.ds(s * RPT, RPT)
        pltpu.sync_copy(acc_msg.at[sl], msg_out.at[sl])
        pltpu.sync_copy(acc_cnt.at[sl], cnt_out.at[sl])

    c = lax.axis_index("c")

    @pl.when(c == 0)
    def _():
        run(hs_ref, 0, 1, msg_d_ref, cnt_d_ref)

    @pl.when(c == 1)
    def _():
        run(hd_ref, 1, 0, msg_s_ref, cnt_s_ref)


def _combine_body(xs_ref, xd_ref, hs_ref, hd_ref,
                  msg_d_ref, cnt_d_ref, msg_s_ref, cnt_s_ref,
                  wss_ref, wns_ref, wsd_ref, wnd_ref,
                  ys_ref, yd_ref):
    inv_d = 1.0 / jnp.maximum(cnt_d_ref[:, :1], 1.0)
    inv_s = 1.0 / jnp.maximum(cnt_s_ref[:, :1], 1.0)
    f32 = jnp.float32
    yd = jnp.dot(hd_ref[...], wsd_ref[...], preferred_element_type=f32)
    yd += jnp.dot(msg_d_ref[...], wnd_ref[...], preferred_element_type=f32) * inv_d
    yd_ref[...] = xd_ref[...] + yd
    ys = jnp.dot(hs_ref[...], wss_ref[...], preferred_element_type=f32)
    ys += jnp.dot(msg_s_ref[...], wns_ref[...], preferred_element_type=f32) * inv_s
    ys_ref[...] = xs_ref[...] + ys


def _make_sc_agg():
    mesh = plsc.VectorSubcoreMesh(core_axis_name="c", subcore_axis_name="s")
    return pl.kernel(
        _sc_agg_body,
        out_type=(
            jax.ShapeDtypeStruct((NPAD, D), jnp.float32),
            jax.ShapeDtypeStruct((NPAD, CW), jnp.float32),
            jax.ShapeDtypeStruct((NPAD, D), jnp.float32),
            jax.ShapeDtypeStruct((NPAD, CW), jnp.float32),
        ),
        mesh=mesh,
        compiler_params=pltpu.CompilerParams(use_tc_tiling_on_sc=False),
        scratch_types=[
            pltpu.VMEM_SHARED((NPAD, D), jnp.float32),
            pltpu.VMEM_SHARED((NPAD, CW), jnp.float32),
            pltpu.VMEM((IC, B), jnp.int32),
            pltpu.VMEM((IC, B), jnp.int32),
            pltpu.VMEM((3, B, D), jnp.float32),
            pltpu.VMEM((B, CW), jnp.float32),
            pltpu.SemaphoreType.DMA,
            pltpu.SemaphoreType.DMA,
            pltpu.SemaphoreType.DMA,
            pltpu.SemaphoreType.DMA,
        ],
    )


def kernel(x_src, x_dst, ln_g_src, ln_b_src, ln_g_dst, ln_b_dst,
           W_self_src, W_nbr_src, W_self_dst, W_nbr_dst, edge_index):
    f32 = jnp.float32
    i32 = jnp.int32
    RB = 2000
    G = N // RB

    gs = ln_g_src.reshape(1, D)
    bs = ln_b_src.reshape(1, D)
    gd = ln_g_dst.reshape(1, D)
    bd = ln_b_dst.reshape(1, D)

    row_spec = pl.BlockSpec((RB, D), lambda i: (i, 0))
    vec_spec = pl.BlockSpec((1, D), lambda i: (0, 0))
    h_src, h_dst = pl.pallas_call(
        _ln_relu_body,
        grid=(G,),
        in_specs=[row_spec, row_spec, vec_spec, vec_spec, vec_spec, vec_spec],
        out_specs=[row_spec, row_spec],
        out_shape=[jax.ShapeDtypeStruct((N, D), f32)] * 2,
    )(x_src, x_dst, gs, bs, gd, bd)

    eis = edge_index.reshape(2, NS, NCH, IC, B)

    z_rows = jnp.zeros((RPT, D), f32)
    z_cnt = jnp.zeros((RPT, CW), f32)
    one_cnt = jnp.ones((B, CW), f32)
    msg_d, cnt_d, msg_s, cnt_s = _make_sc_agg()(
        h_src, h_dst, eis, z_rows, z_cnt, one_cnt)

    cnt_spec = pl.BlockSpec((RB, CW), lambda i: (i, 0))
    w_spec = pl.BlockSpec((D, D), lambda i: (0, 0))
    y_src, y_dst = pl.pallas_call(
        _combine_body,
        grid=(G,),
        in_specs=[row_spec, row_spec, row_spec, row_spec,
                  row_spec, cnt_spec, row_spec, cnt_spec,
                  w_spec, w_spec, w_spec, w_spec],
        out_specs=[row_spec, row_spec],
        out_shape=[jax.ShapeDtypeStruct((N, D), f32)] * 2,
    )(x_src, x_dst, h_src, h_dst,
      msg_d, cnt_d, msg_s, cnt_s,
      W_self_src, W_nbr_src, W_self_dst, W_nbr_dst)

    return (y_src, y_dst)


# speedup vs baseline: 2.2912x; 1.1335x over previous
import jax
import jax.numpy as jnp
from jax import lax
from jax.experimental import pallas as pl
from jax.experimental.pallas import tpu as pltpu
from jax.experimental.pallas import tpu_sc as plsc

N = 10000
E = 320000
D = 128
NS = 16
EPT = E // NS
B = 80
NBT = 250
EPTP = NBT * B
IC = 25
NCH = NBT // IC
NPAD = 10240
RPT = NPAD // NS
CW = 16
PAD_ROW = 10100


def _ln_relu_body(xs_ref, xd_ref, gs_ref, bs_ref, gd_ref, bd_ref,
                  hs_ref, hd_ref):
    for x_ref, g_ref, b_ref, h_ref in (
        (xs_ref, gs_ref, bs_ref, hs_ref),
        (xd_ref, gd_ref, bd_ref, hd_ref),
    ):
        x = x_ref[...]
        m = jnp.mean(x, axis=-1, keepdims=True)
        v = jnp.mean(jnp.square(x - m), axis=-1, keepdims=True)
        h = (x - m) * jax.lax.rsqrt(v + 1e-5) * g_ref[...] + b_ref[...]
        h_ref[...] = jnp.maximum(h, 0.0)


def _sc_agg_body(hs_ref, hd_ref, ei_ref, z_rows_ref, z_cnt_ref, one_cnt_ref,
                 msg_d_ref, cnt_d_ref, msg_s_ref, cnt_s_ref,
                 acc_msg, acc_cnt, gidx, sidx, rows, ones,
                 sem_g0, sem_g1, sem_s0, sem_s1):
    s = lax.axis_index("s")
    sem_g = (sem_g0, sem_g1)
    sem_s = (sem_s0, sem_s1)

    def run(h_ref, gsel, ssel, msg_out, cnt_out):
        pltpu.sync_copy(one_cnt_ref, ones)
        pltpu.sync_copy(z_rows_ref, acc_msg.at[pl.ds(s * RPT, RPT)])
        pltpu.sync_copy(z_cnt_ref, acc_cnt.at[pl.ds(s * RPT, RPT)])
        plsc.subcore_barrier()

        def gather(k, b):
            return pltpu.async_copy(
                h_ref.at[gidx.at[k]], rows.at[b], sem_g[b % 2])

        def chunk(ci, carry):
            pltpu.sync_copy(ei_ref.at[gsel, s, ci], gidx)
            pltpu.sync_copy(ei_ref.at[ssel, s, ci], sidx)
            g = {0: gather(0, 0), 1: gather(1, 1), 2: gather(2, 2)}
            sc = {}
            cnt_dmas = []
            for k in range(IC):
                b = k % 3
                g.pop(k).wait()
                sc[k] = pltpu.async_copy(
                    rows.at[b], acc_msg.at[sidx.at[k]], sem_s[0], add=True)
                cnt_dmas.append(pltpu.async_copy(
                    ones, acc_cnt.at[sidx.at[k]], sem_s[1], add=True))
                if k >= 1 and k + 2 < IC:
                    sc.pop(k - 1).wait()
                    g[k + 2] = gather(k + 2, (k + 2) % 3)
            for kk in sorted(sc):
                sc[kk].wait()
            for d in cnt_dmas:
                d.wait()
            return carry

        lax.fori_loop(0, NCH, chunk, 0)
        plsc.subcore_barrier()
        sl = pl.ds(s * RPT, RPT)
        pltpu.sync_copy(acc_msg.at[sl], msg_out.at[sl])
        pltpu.sync_copy(acc_cnt.at[sl], cnt_out.at[sl])

    c = lax.axis_index("c")

    @pl.when(c == 0)
    def _():
        run(hs_ref, 0, 1, msg_d_ref, cnt_d_ref)

    @pl.when(c == 1)
    def _():
        run(hd_ref, 1, 0, msg_s_ref, cnt_s_ref)


def _combine_body(xs_ref, xd_ref, hs_ref, hd_ref,
                  msg_d_ref, cnt_d_ref, msg_s_ref, cnt_s_ref,
                  wss_ref, wns_ref, wsd_ref, wnd_ref,
                  ys_ref, yd_ref):
    inv_d = 1.0 / jnp.maximum(cnt_d_ref[:, :1], 1.0)
    inv_s = 1.0 / jnp.maximum(cnt_s_ref[:, :1], 1.0)
    f32 = jnp.float32
    yd = jnp.dot(hd_ref[...], wsd_ref[...], preferred_element_type=f32)
    yd += jnp.dot(msg_d_ref[...], wnd_ref[...], preferred_element_type=f32) * inv_d
    yd_ref[...] = xd_ref[...] + yd
    ys = jnp.dot(hs_ref[...], wss_ref[...], preferred_element_type=f32)
    ys += jnp.dot(msg_s_ref[...], wns_ref[...], preferred_element_type=f32) * inv_s
    ys_ref[...] = xs_ref[...] + ys


def _make_sc_agg():
    mesh = plsc.VectorSubcoreMesh(core_axis_name="c", subcore_axis_name="s")
    return pl.kernel(
        _sc_agg_body,
        out_type=(
            jax.ShapeDtypeStruct((NPAD, D), jnp.float32),
            jax.ShapeDtypeStruct((NPAD, CW), jnp.float32),
            jax.ShapeDtypeStruct((NPAD, D), jnp.float32),
            jax.ShapeDtypeStruct((NPAD, CW), jnp.float32),
        ),
        mesh=mesh,
        compiler_params=pltpu.CompilerParams(use_tc_tiling_on_sc=False),
        scratch_types=[
            pltpu.VMEM_SHARED((NPAD, D), jnp.float32),
            pltpu.VMEM_SHARED((NPAD, CW), jnp.float32),
            pltpu.VMEM((IC, B), jnp.int32),
            pltpu.VMEM((IC, B), jnp.int32),
            pltpu.VMEM((3, B, D), jnp.float32),
            pltpu.VMEM((B, CW), jnp.float32),
            pltpu.SemaphoreType.DMA,
            pltpu.SemaphoreType.DMA,
            pltpu.SemaphoreType.DMA,
            pltpu.SemaphoreType.DMA,
        ],
    )


def kernel(x_src, x_dst, ln_g_src, ln_b_src, ln_g_dst, ln_b_dst,
           W_self_src, W_nbr_src, W_self_dst, W_nbr_dst, edge_index):
    f32 = jnp.float32
    i32 = jnp.int32
    RB = 2000
    G = N // RB

    gs = ln_g_src.reshape(1, D)
    bs = ln_b_src.reshape(1, D)
    gd = ln_g_dst.reshape(1, D)
    bd = ln_b_dst.reshape(1, D)

    row_spec = pl.BlockSpec((RB, D), lambda i: (i, 0))
    vec_spec = pl.BlockSpec((1, D), lambda i: (0, 0))
    h_src, h_dst = pl.pallas_call(
        _ln_relu_body,
        grid=(G,),
        in_specs=[row_spec, row_spec, vec_spec, vec_spec, vec_spec, vec_spec],
        out_specs=[row_spec, row_spec],
        out_shape=[jax.ShapeDtypeStruct((N, D), f32)] * 2,
    )(x_src, x_dst, gs, bs, gd, bd)

    eis = edge_index.reshape(2, NS, NCH, IC, B)

    z_rows = jnp.zeros((RPT, D), f32)
    z_cnt = jnp.zeros((RPT, CW), f32)
    one_cnt = jnp.ones((B, CW), f32)
    msg_d, cnt_d, msg_s, cnt_s = _make_sc_agg()(
        h_src, h_dst, eis, z_rows, z_cnt, one_cnt)

    cnt_spec = pl.BlockSpec((RB, CW), lambda i: (i, 0))
    w_spec = pl.BlockSpec((D, D), lambda i: (0, 0))
    y_src, y_dst = pl.pallas_call(
        _combine_body,
        grid=(G,),
        in_specs=[row_spec, row_spec, row_spec, row_spec,
                  row_spec, cnt_spec, row_spec, cnt_spec,
                  w_spec, w_spec, w_spec, w_spec],
        out_specs=[row_spec, row_spec],
        out_shape=[jax.ShapeDtypeStruct((N, D), f32)] * 2,
    )(x_src, x_dst, h_src, h_dst,
      msg_d, cnt_d, msg_s, cnt_s,
      W_self_src, W_nbr_src, W_self_dst, W_nbr_dst)

    return (y_src, y_dst)


# speedup vs baseline: 2.3844x; 1.0407x over previous
import jax
import jax.numpy as jnp
from jax import lax
from jax.experimental import pallas as pl
from jax.experimental.pallas import tpu as pltpu
from jax.experimental.pallas import tpu_sc as plsc

N = 10000
E = 320000
D = 128
NS = 16
EPT = E // NS
B = 80
NBT = 250
EPTP = NBT * B
IC = 25
NCH = NBT // IC
NPAD = 10112
RPT = NPAD // NS
CW = 16
PAD_ROW = 10100


def _ln_relu_body(xs_ref, xd_ref, gs_ref, bs_ref, gd_ref, bd_ref,
                  hs_ref, hd_ref):
    for x_ref, g_ref, b_ref, h_ref in (
        (xs_ref, gs_ref, bs_ref, hs_ref),
        (xd_ref, gd_ref, bd_ref, hd_ref),
    ):
        x = x_ref[...]
        m = jnp.mean(x, axis=-1, keepdims=True)
        v = jnp.mean(jnp.square(x - m), axis=-1, keepdims=True)
        h = (x - m) * jax.lax.rsqrt(v + 1e-5) * g_ref[...] + b_ref[...]
        h_ref[...] = jnp.maximum(h, 0.0)


def _sc_agg_body(hs_ref, hd_ref, ei_ref, z_rows_ref, z_cnt_ref, one_cnt_ref,
                 msg_d_ref, cnt_d_ref, msg_s_ref, cnt_s_ref,
                 acc_msg, acc_cnt, gidx, sidx, rows, ones,
                 sem_g0, sem_g1, sem_s0, sem_s1, sem_i):
    s = lax.axis_index("s")
    sem_g = (sem_g0, sem_g1)
    sem_s = (sem_s0, sem_s1)

    def run(h_ref, gsel, ssel, msg_out, cnt_out):
        pltpu.sync_copy(one_cnt_ref, ones)
        pltpu.sync_copy(z_rows_ref, acc_msg.at[pl.ds(s * RPT, RPT)])
        pltpu.sync_copy(z_cnt_ref, acc_cnt.at[pl.ds(s * RPT, RPT)])
        plsc.subcore_barrier()

        def gather(sl, k, b):
            return pltpu.async_copy(
                h_ref.at[gidx.at[sl, k]], rows.at[b], sem_g[b % 2])

        pltpu.sync_copy(ei_ref.at[gsel, s, 0], gidx.at[0])
        pltpu.sync_copy(ei_ref.at[ssel, s, 0], sidx.at[0])

        def chunk(ci, carry):
            sl = lax.rem(ci, 2)
            nsl = 1 - sl

            @pl.when(ci > 0)
            def _():
                pltpu.make_async_copy(
                    ei_ref.at[gsel, s, ci], gidx.at[sl], sem_i).wait()
                pltpu.make_async_copy(
                    ei_ref.at[ssel, s, ci], sidx.at[sl], sem_i).wait()

            @pl.when(ci < NCH - 1)
            def _():
                pltpu.async_copy(
                    ei_ref.at[gsel, s, ci + 1], gidx.at[nsl], sem_i)
                pltpu.async_copy(
                    ei_ref.at[ssel, s, ci + 1], sidx.at[nsl], sem_i)

            g = {0: gather(sl, 0, 0), 1: gather(sl, 1, 1), 2: gather(sl, 2, 2)}
            sc = {}
            cnt_dmas = []
            for k in range(IC):
                b = k % 3
                g.pop(k).wait()
                sc[k] = pltpu.async_copy(
                    rows.at[b], acc_msg.at[sidx.at[sl, k]], sem_s[0], add=True)
                cnt_dmas.append(pltpu.async_copy(
                    ones, acc_cnt.at[sidx.at[sl, k]], sem_s[1], add=True))
                if k >= 1 and k + 2 < IC:
                    sc.pop(k - 1).wait()
                    g[k + 2] = gather(sl, k + 2, (k + 2) % 3)
            for kk in sorted(sc):
                sc[kk].wait()
            for d in cnt_dmas:
                d.wait()
            return carry

        lax.fori_loop(0, NCH, chunk, 0)
        plsc.subcore_barrier()
        sl = pl.ds(s * RPT, RPT)
        pltpu.sync_copy(acc_msg.at[sl], msg_out.at[sl])
        pltpu.sync_copy(acc_cnt.at[sl], cnt_out.at[sl])

    c = lax.axis_index("c")

    @pl.when(c == 0)
    def _():
        run(hs_ref, 0, 1, msg_d_ref, cnt_d_ref)

    @pl.when(c == 1)
    def _():
        run(hd_ref, 1, 0, msg_s_ref, cnt_s_ref)


def _combine_body(xs_ref, xd_ref, hs_ref, hd_ref,
                  msg_d_ref, cnt_d_ref, msg_s_ref, cnt_s_ref,
                  wss_ref, wns_ref, wsd_ref, wnd_ref,
                  ys_ref, yd_ref):
    inv_d = 1.0 / jnp.maximum(cnt_d_ref[:, :1], 1.0)
    inv_s = 1.0 / jnp.maximum(cnt_s_ref[:, :1], 1.0)
    f32 = jnp.float32
    yd = jnp.dot(hd_ref[...], wsd_ref[...], preferred_element_type=f32)
    yd += jnp.dot(msg_d_ref[...], wnd_ref[...], preferred_element_type=f32) * inv_d
    yd_ref[...] = xd_ref[...] + yd
    ys = jnp.dot(hs_ref[...], wss_ref[...], preferred_element_type=f32)
    ys += jnp.dot(msg_s_ref[...], wns_ref[...], preferred_element_type=f32) * inv_s
    ys_ref[...] = xs_ref[...] + ys


def _make_sc_agg():
    mesh = plsc.VectorSubcoreMesh(core_axis_name="c", subcore_axis_name="s")
    return pl.kernel(
        _sc_agg_body,
        out_type=(
            jax.ShapeDtypeStruct((NPAD, D), jnp.float32),
            jax.ShapeDtypeStruct((NPAD, CW), jnp.float32),
            jax.ShapeDtypeStruct((NPAD, D), jnp.float32),
            jax.ShapeDtypeStruct((NPAD, CW), jnp.float32),
        ),
        mesh=mesh,
        compiler_params=pltpu.CompilerParams(use_tc_tiling_on_sc=False),
        scratch_types=[
            pltpu.VMEM_SHARED((NPAD, D), jnp.float32),
            pltpu.VMEM_SHARED((NPAD, CW), jnp.float32),
            pltpu.VMEM((2, IC, B), jnp.int32),
            pltpu.VMEM((2, IC, B), jnp.int32),
            pltpu.VMEM((3, B, D), jnp.float32),
            pltpu.VMEM((B, CW), jnp.float32),
            pltpu.SemaphoreType.DMA,
            pltpu.SemaphoreType.DMA,
            pltpu.SemaphoreType.DMA,
            pltpu.SemaphoreType.DMA,
            pltpu.SemaphoreType.DMA,
        ],
    )


def kernel(x_src, x_dst, ln_g_src, ln_b_src, ln_g_dst, ln_b_dst,
           W_self_src, W_nbr_src, W_self_dst, W_nbr_dst, edge_index):
    f32 = jnp.float32
    i32 = jnp.int32
    RB = 2000
    G = N // RB

    gs = ln_g_src.reshape(1, D)
    bs = ln_b_src.reshape(1, D)
    gd = ln_g_dst.reshape(1, D)
    bd = ln_b_dst.reshape(1, D)

    row_spec = pl.BlockSpec((RB, D), lambda i: (i, 0))
    vec_spec = pl.BlockSpec((1, D), lambda i: (0, 0))
    h_src, h_dst = pl.pallas_call(
        _ln_relu_body,
        grid=(G,),
        in_specs=[row_spec, row_spec, vec_spec, vec_spec, vec_spec, vec_spec],
        out_specs=[row_spec, row_spec],
        out_shape=[jax.ShapeDtypeStruct((N, D), f32)] * 2,
    )(x_src, x_dst, gs, bs, gd, bd)

    eis = edge_index.reshape(2, NS, NCH, IC, B)

    z_rows = jnp.zeros((RPT, D), f32)
    z_cnt = jnp.zeros((RPT, CW), f32)
    one_cnt = jnp.ones((B, CW), f32)
    msg_d, cnt_d, msg_s, cnt_s = _make_sc_agg()(
        h_src, h_dst, eis, z_rows, z_cnt, one_cnt)

    cnt_spec = pl.BlockSpec((RB, CW), lambda i: (i, 0))
    w_spec = pl.BlockSpec((D, D), lambda i: (0, 0))
    y_src, y_dst = pl.pallas_call(
        _combine_body,
        grid=(G,),
        in_specs=[row_spec, row_spec, row_spec, row_spec,
                  row_spec, cnt_spec, row_spec, cnt_spec,
                  w_spec, w_spec, w_spec, w_spec],
        out_specs=[row_spec, row_spec],
        out_shape=[jax.ShapeDtypeStruct((N, D), f32)] * 2,
    )(x_src, x_dst, h_src, h_dst,
      msg_d, cnt_d, msg_s, cnt_s,
      W_self_src, W_nbr_src, W_self_dst, W_nbr_dst)

    return (y_src, y_dst)
